# baseline (device time: 10653 ns/iter reference)
import jax
import jax.numpy as jnp
from jax import lax
from jax.experimental import pallas as pl
from jax.experimental.pallas import tpu as pltpu

N_DEV = 4
N_RDMA = 6
K = 192


def _body(x_ref, d_ref, out_ref,
          pad_r_buf, pad_l_buf, pad_d_buf,
          fl_buf, fr_buf, diag_buf,
          da, db, dd,
          send_sems, recv_sems):
    me = lax.axis_index("i")
    left = (me - 1) % N_DEV
    right = (me + 1) % N_DEV
    diag = (me + 2) % N_DEV

    m_per = x_ref.shape[0]

    barrier_sem = pltpu.get_barrier_semaphore()
    for nbr in (left, right, diag):
        pl.semaphore_signal(
            barrier_sem, inc=1,
            device_id=(nbr,), device_id_type=pl.DeviceIdType.MESH,
        )

    xv = x_ref[:, :].astype(jnp.bfloat16)
    dv = d_ref[:].reshape(1, m_per)
    tri = (
        lax.broadcasted_iota(jnp.int32, (m_per, m_per), 0)
        <= lax.broadcasted_iota(jnp.int32, (m_per, m_per), 1)
    ).astype(jnp.bfloat16)
    k_iota = lax.broadcasted_iota(jnp.int32, (K, m_per), 0)

    ids4 = (me + lax.broadcasted_iota(jnp.int32, (N_DEV, 1), 0)) % N_DEV
    mask4 = dv == ids4
    incl4 = jnp.dot(mask4.astype(jnp.bfloat16), tri,
                    preferred_element_type=jnp.float32)

    def pack(r):
        mb = mask4[r:r + 1, :]
        pos = (incl4[r:r + 1, :] - 1.0).astype(jnp.int32)
        sel = jnp.logical_and(k_iota == pos, mb).astype(jnp.bfloat16)
        return jnp.dot(sel, xv,
                       preferred_element_type=jnp.float32).astype(jnp.bfloat16)

    pad_d_buf[:, :] = pack(2)

    pl.semaphore_wait(barrier_sem, 3)

    def mk(idx, src, dst, dev):
        return pltpu.make_async_remote_copy(
            src_ref=src, dst_ref=dst,
            send_sem=send_sems.at[idx], recv_sem=recv_sems.at[idx],
            device_id=(dev,), device_id_type=pl.DeviceIdType.MESH,
        )

    r0 = mk(0, d_ref, da, right)
    r1 = mk(1, d_ref, db, left)
    r2 = mk(2, d_ref, dd, diag)
    r5 = mk(5, pad_d_buf, diag_buf, diag)
    r0.start()
    r1.start()
    r2.start()
    r5.start()
    pad_r_buf[:, :] = pack(1)
    r3 = mk(3, pad_r_buf, fl_buf, right)
    r3.start()
    pad_l_buf[:, :] = pack(3)
    r4 = mk(4, pad_l_buf, fr_buf, left)
    r4.start()

    r0.wait()
    r1.wait()
    r2.wait()

    dall = jnp.concatenate(
        [dv, db[:].reshape(1, m_per), dd[:].reshape(1, m_per),
         da[:].reshape(1, m_per)], axis=0)
    maskr = dall == me
    cnt4 = jnp.sum(maskr.astype(jnp.float32), axis=1)
    gids = [me, right, diag, left]
    cnts = [cnt4[r] for r in range(N_DEV)]
    offs = []
    for r in range(N_DEV):
        off = jnp.float32(0.0)
        for rp in range(N_DEV):
            if rp != r:
                off = off + jnp.where(gids[rp] < gids[r], cnts[rp], 0.0)
        offs.append(off)

    row_iota = lax.broadcasted_iota(jnp.int32, (m_per, K), 0)
    col_iota = lax.broadcasted_iota(jnp.int32, (m_per, K), 1)

    def place(r):
        off_i = offs[r].astype(jnp.int32)
        cnt_i = cnts[r].astype(jnp.int32)
        q = jnp.logical_and(row_iota - col_iota == off_i, col_iota < cnt_i)
        return q.astype(jnp.bfloat16)

    sq_iota = lax.broadcasted_iota(jnp.int32, (m_per, m_per), 0)
    pos0 = (offs[0] + incl4[0:1, :] - 1.0).astype(jnp.int32)
    p_own = jnp.logical_and(sq_iota == pos0, maskr[0:1, :]).astype(jnp.bfloat16)
    acc = jnp.dot(p_own, xv, preferred_element_type=jnp.float32)

    r4.wait()
    acc = acc + jnp.dot(place(1), fr_buf[:, :],
                        preferred_element_type=jnp.float32)
    r3.wait()
    acc = acc + jnp.dot(place(3), fl_buf[:, :],
                        preferred_element_type=jnp.float32)
    q_diag = place(2)
    r5.wait()
    acc = acc + jnp.dot(q_diag, diag_buf[:, :],
                        preferred_element_type=jnp.float32)

    out_ref[:, :] = acc.astype(jnp.bfloat16)


def kernel(x, dest):
    m_per, n = x.shape

    return pl.pallas_call(
        _body,
        out_shape=jax.ShapeDtypeStruct((m_per, n), jnp.bfloat16),
        in_specs=[
            pl.BlockSpec(memory_space=pltpu.VMEM),
            pl.BlockSpec(memory_space=pltpu.VMEM),
        ],
        out_specs=pl.BlockSpec(memory_space=pltpu.VMEM),
        scratch_shapes=[
            pltpu.VMEM((K, n), jnp.bfloat16),
            pltpu.VMEM((K, n), jnp.bfloat16),
            pltpu.VMEM((K, n), jnp.bfloat16),
            pltpu.VMEM((K, n), jnp.bfloat16),
            pltpu.VMEM((K, n), jnp.bfloat16),
            pltpu.VMEM((K, n), jnp.bfloat16),
            pltpu.VMEM((m_per,), jnp.int32),
            pltpu.VMEM((m_per,), jnp.int32),
            pltpu.VMEM((m_per,), jnp.int32),
            pltpu.SemaphoreType.DMA((N_RDMA,)),
            pltpu.SemaphoreType.DMA((N_RDMA,)),
        ],
        compiler_params=pltpu.CompilerParams(collective_id=0),
    )(x, dest)


# device time: 10446 ns/iter; 1.0198x vs baseline; 1.0198x over previous
import jax
import jax.numpy as jnp
from jax import lax
from jax.experimental import pallas as pl
from jax.experimental.pallas import tpu as pltpu

N_DEV = 4
N_RDMA = 6
K = 192


def _body(x_ref, d_ref, out_ref,
          pad_r_buf, pad_l_buf, pad_d_buf,
          fl_buf, fr_buf, diag_buf,
          da, db, dd,
          send_sems, recv_sems):
    me = lax.axis_index("i")
    left = (me - 1) % N_DEV
    right = (me + 1) % N_DEV
    diag = (me + 2) % N_DEV

    m_per = x_ref.shape[0]

    barrier_sem = pltpu.get_barrier_semaphore()
    for nbr in (left, right, diag):
        pl.semaphore_signal(
            barrier_sem, inc=1,
            device_id=(nbr,), device_id_type=pl.DeviceIdType.MESH,
        )

    xv = x_ref[:, :].astype(jnp.bfloat16)
    dv = d_ref[:].reshape(1, m_per)
    tri = (
        lax.broadcasted_iota(jnp.int32, (m_per, m_per), 0)
        <= lax.broadcasted_iota(jnp.int32, (m_per, m_per), 1)
    ).astype(jnp.bfloat16)
    k_iota = lax.broadcasted_iota(jnp.int32, (K, m_per), 0)

    ids4 = (me + lax.broadcasted_iota(jnp.int32, (N_DEV, 1), 0)) % N_DEV
    mask4 = dv == ids4
    incl4 = jnp.dot(mask4.astype(jnp.bfloat16), tri,
                    preferred_element_type=jnp.float32)

    def pack(r):
        mb = mask4[r:r + 1, :]
        pos = (incl4[r:r + 1, :] - 1.0).astype(jnp.int32)
        sel = jnp.logical_and(k_iota == pos, mb).astype(jnp.bfloat16)
        return jnp.dot(sel, xv,
                       preferred_element_type=jnp.float32).astype(jnp.bfloat16)

    pad_d_buf[:, :] = pack(2)
    pad_r_buf[:, :] = pack(1)
    pad_l_buf[:, :] = pack(3)

    pl.semaphore_wait(barrier_sem, 3)

    def mk(idx, src, dst, dev):
        return pltpu.make_async_remote_copy(
            src_ref=src, dst_ref=dst,
            send_sem=send_sems.at[idx], recv_sem=recv_sems.at[idx],
            device_id=(dev,), device_id_type=pl.DeviceIdType.MESH,
        )

    r0 = mk(0, d_ref, da, right)
    r1 = mk(1, d_ref, db, left)
    r2 = mk(2, d_ref, dd, diag)
    r5 = mk(5, pad_d_buf, diag_buf, diag)
    r3 = mk(3, pad_r_buf, fl_buf, right)
    r4 = mk(4, pad_l_buf, fr_buf, left)
    r0.start()
    r1.start()
    r2.start()
    r5.start()
    r3.start()
    r4.start()

    r0.wait()
    r1.wait()
    r2.wait()

    dall = jnp.concatenate(
        [dv, db[:].reshape(1, m_per), dd[:].reshape(1, m_per),
         da[:].reshape(1, m_per)], axis=0)
    maskr = dall == me
    cnt4 = jnp.sum(maskr.astype(jnp.float32), axis=1)
    gids = [me, right, diag, left]
    cnts = [cnt4[r] for r in range(N_DEV)]
    offs = []
    for r in range(N_DEV):
        off = jnp.float32(0.0)
        for rp in range(N_DEV):
            if rp != r:
                off = off + jnp.where(gids[rp] < gids[r], cnts[rp], 0.0)
        offs.append(off)

    row_iota = lax.broadcasted_iota(jnp.int32, (m_per, K), 0)
    col_iota = lax.broadcasted_iota(jnp.int32, (m_per, K), 1)

    def place(r):
        off_i = offs[r].astype(jnp.int32)
        cnt_i = cnts[r].astype(jnp.int32)
        q = jnp.logical_and(row_iota - col_iota == off_i, col_iota < cnt_i)
        return q.astype(jnp.bfloat16)

    sq_iota = lax.broadcasted_iota(jnp.int32, (m_per, m_per), 0)
    pos0 = (offs[0] + incl4[0:1, :] - 1.0).astype(jnp.int32)
    p_own = jnp.logical_and(sq_iota == pos0, maskr[0:1, :]).astype(jnp.bfloat16)
    acc = jnp.dot(p_own, xv, preferred_element_type=jnp.float32)

    r4.wait()
    acc = acc + jnp.dot(place(1), fr_buf[:, :],
                        preferred_element_type=jnp.float32)
    r3.wait()
    acc = acc + jnp.dot(place(3), fl_buf[:, :],
                        preferred_element_type=jnp.float32)
    q_diag = place(2)
    r5.wait()
    acc = acc + jnp.dot(q_diag, diag_buf[:, :],
                        preferred_element_type=jnp.float32)

    out_ref[:, :] = acc.astype(jnp.bfloat16)


def kernel(x, dest):
    m_per, n = x.shape

    return pl.pallas_call(
        _body,
        out_shape=jax.ShapeDtypeStruct((m_per, n), jnp.bfloat16),
        in_specs=[
            pl.BlockSpec(memory_space=pltpu.VMEM),
            pl.BlockSpec(memory_space=pltpu.VMEM),
        ],
        out_specs=pl.BlockSpec(memory_space=pltpu.VMEM),
        scratch_shapes=[
            pltpu.VMEM((K, n), jnp.bfloat16),
            pltpu.VMEM((K, n), jnp.bfloat16),
            pltpu.VMEM((K, n), jnp.bfloat16),
            pltpu.VMEM((K, n), jnp.bfloat16),
            pltpu.VMEM((K, n), jnp.bfloat16),
            pltpu.VMEM((K, n), jnp.bfloat16),
            pltpu.VMEM((m_per,), jnp.int32),
            pltpu.VMEM((m_per,), jnp.int32),
            pltpu.VMEM((m_per,), jnp.int32),
            pltpu.SemaphoreType.DMA((N_RDMA,)),
            pltpu.SemaphoreType.DMA((N_RDMA,)),
        ],
        compiler_params=pltpu.CompilerParams(collective_id=0),
    )(x, dest)
